# baseline (speedup 1.0000x reference)
import jax
import jax.numpy as jnp
from jax.experimental import pallas as pl
from jax.experimental.pallas import tpu as pltpu

N = 4096
D = 512
H = 8
DH = D // H

ROW_BLK = 512


def _ln(h, g, b):
    mu = jnp.mean(h, axis=-1, keepdims=True)
    var = jnp.mean((h - mu) ** 2, axis=-1, keepdims=True)
    return (h - mu) / jnp.sqrt(var + 1e-5) * g + b


def _qkv_body(x_ref, g_ref, b_ref, wq_ref, bq_ref, wk_ref, bk_ref, wv_ref, bv_ref,
              q_ref, k_ref, v_ref):
    xn = _ln(x_ref[...], g_ref[...], b_ref[...])
    q_ref[...] = jnp.dot(xn, wq_ref[...], preferred_element_type=jnp.float32) + bq_ref[...]
    k_ref[...] = jnp.dot(xn, wk_ref[...], preferred_element_type=jnp.float32) + bk_ref[...]
    v_ref[...] = jnp.dot(xn, wv_ref[...], preferred_element_type=jnp.float32) + bv_ref[...]


def _qkv(x, ln1_g, ln1_b, Wq, bq, Wk, bk, Wv, bv):
    grid = (N // ROW_BLK,)
    row_spec = pl.BlockSpec((ROW_BLK, D), lambda i: (i, 0))
    full_w = pl.BlockSpec((D, D), lambda i: (0, 0))
    full_b = pl.BlockSpec((D,), lambda i: (0,))
    return pl.pallas_call(
        _qkv_body,
        grid=grid,
        in_specs=[row_spec, full_b, full_b, full_w, full_b, full_w, full_b, full_w, full_b],
        out_specs=[row_spec, row_spec, row_spec],
        out_shape=[jax.ShapeDtypeStruct((N, D), jnp.float32)] * 3,
    )(x, ln1_g, ln1_b, Wq, bq, Wk, bk, Wv, bv)


def _tail_body(x_ref, ao_ref, wo_ref, bo_ref, g_ref, b_ref, w1_ref, b1_ref, w2_ref, b2_ref,
               out_ref):
    h = x_ref[...] + jnp.dot(ao_ref[...], wo_ref[...], preferred_element_type=jnp.float32) + bo_ref[...]
    hn = _ln(h, g_ref[...], b_ref[...])
    up = jax.nn.gelu(jnp.dot(hn, w1_ref[...], preferred_element_type=jnp.float32) + b1_ref[...])
    out_ref[...] = h + jnp.dot(up, w2_ref[...], preferred_element_type=jnp.float32) + b2_ref[...]


def _tail(x, attnout, Wo, bo, ln2_g, ln2_b, W1, b1, W2, b2):
    grid = (N // ROW_BLK,)
    row_spec = pl.BlockSpec((ROW_BLK, D), lambda i: (i, 0))
    return pl.pallas_call(
        _tail_body,
        grid=grid,
        in_specs=[row_spec, row_spec,
                  pl.BlockSpec((D, D), lambda i: (0, 0)),
                  pl.BlockSpec((D,), lambda i: (0,)),
                  pl.BlockSpec((D,), lambda i: (0,)),
                  pl.BlockSpec((D,), lambda i: (0,)),
                  pl.BlockSpec((D, 2 * D), lambda i: (0, 0)),
                  pl.BlockSpec((2 * D,), lambda i: (0,)),
                  pl.BlockSpec((2 * D, D), lambda i: (0, 0)),
                  pl.BlockSpec((D,), lambda i: (0,))],
        out_specs=row_spec,
        out_shape=jax.ShapeDtypeStruct((N, D), jnp.float32),
    )(x, attnout, Wo, bo, ln2_g, ln2_b, W1, b1, W2, b2)


def kernel(x, batch_mask_indices, batch_mask_values, attention_mask_indices, attention_mask_values, ln1_g, ln1_b, Wq, bq, Wk, bk, Wv, bv, Wo, bo, ln2_g, ln2_b, W1, b1, W2, b2):
    rows = attention_mask_indices[0]
    cols = attention_mask_indices[1]
    q, k, v = _qkv(x, ln1_g, ln1_b, Wq, bq, Wk, bk, Wv, bv)
    qh = q.reshape(N, H, DH)
    kh = k.reshape(N, H, DH)
    vh = v.reshape(N, H, DH)
    scores = jnp.einsum('ehd,ehd->eh', qh[rows], kh[cols]) / jnp.sqrt(jnp.float32(DH))
    m = jax.ops.segment_max(scores, rows, num_segments=N)
    m = jnp.where(jnp.isfinite(m), m, 0.0)
    w = jnp.exp(scores - m[rows]) * (batch_mask_values * attention_mask_values)[:, None]
    denom = jax.ops.segment_sum(w, rows, num_segments=N) + 1e-9
    attn = w / denom[rows]
    out = jax.ops.segment_sum(attn[:, :, None] * vh[cols], rows, num_segments=N)
    return _tail(x, out.reshape(N, D), Wo, bo, ln2_g, ln2_b, W1, b1, W2, b2)


# speedup vs baseline: 12.1022x; 12.1022x over previous
import jax
import jax.numpy as jnp
from jax.experimental import pallas as pl
from jax.experimental.pallas import tpu as pltpu

N = 4096
D = 512
H = 8
DH = D // H

ROW_BLK = 512


def _ln(h, g, b):
    mu = jnp.mean(h, axis=-1, keepdims=True)
    var = jnp.mean((h - mu) ** 2, axis=-1, keepdims=True)
    return (h - mu) / jnp.sqrt(var + 1e-5) * g + b


def _qkv_body(x_ref, g_ref, b_ref, wq_ref, bq_ref, wk_ref, bk_ref, wv_ref, bv_ref,
              q_ref, k_ref, v_ref):
    xn = _ln(x_ref[...], g_ref[...], b_ref[...])
    q_ref[...] = jnp.dot(xn, wq_ref[...], preferred_element_type=jnp.float32) + bq_ref[...]
    k_ref[...] = jnp.dot(xn, wk_ref[...], preferred_element_type=jnp.float32) + bk_ref[...]
    v_ref[...] = jnp.dot(xn, wv_ref[...], preferred_element_type=jnp.float32) + bv_ref[...]


def _qkv(x, ln1_g, ln1_b, Wq, bq, Wk, bk, Wv, bv):
    grid = (N // ROW_BLK,)
    row_spec = pl.BlockSpec((ROW_BLK, D), lambda i: (i, 0))
    full_w = pl.BlockSpec((D, D), lambda i: (0, 0))
    full_b = pl.BlockSpec((D,), lambda i: (0,))
    return pl.pallas_call(
        _qkv_body,
        grid=grid,
        in_specs=[row_spec, full_b, full_b, full_w, full_b, full_w, full_b, full_w, full_b],
        out_specs=[row_spec, row_spec, row_spec],
        out_shape=[jax.ShapeDtypeStruct((N, D), jnp.float32)] * 3,
    )(x, ln1_g, ln1_b, Wq, bq, Wk, bk, Wv, bv)


def _tail_body(x_ref, ao_ref, wo_ref, bo_ref, g_ref, b_ref, w1_ref, b1_ref, w2_ref, b2_ref,
               out_ref):
    h = x_ref[...] + jnp.dot(ao_ref[...], wo_ref[...], preferred_element_type=jnp.float32) + bo_ref[...]
    hn = _ln(h, g_ref[...], b_ref[...])
    up = jax.nn.gelu(jnp.dot(hn, w1_ref[...], preferred_element_type=jnp.float32) + b1_ref[...])
    out_ref[...] = h + jnp.dot(up, w2_ref[...], preferred_element_type=jnp.float32) + b2_ref[...]


def _tail(x, attnout, Wo, bo, ln2_g, ln2_b, W1, b1, W2, b2):
    grid = (N // ROW_BLK,)
    row_spec = pl.BlockSpec((ROW_BLK, D), lambda i: (i, 0))
    return pl.pallas_call(
        _tail_body,
        grid=grid,
        in_specs=[row_spec, row_spec,
                  pl.BlockSpec((D, D), lambda i: (0, 0)),
                  pl.BlockSpec((D,), lambda i: (0,)),
                  pl.BlockSpec((D,), lambda i: (0,)),
                  pl.BlockSpec((D,), lambda i: (0,)),
                  pl.BlockSpec((D, 2 * D), lambda i: (0, 0)),
                  pl.BlockSpec((2 * D,), lambda i: (0,)),
                  pl.BlockSpec((2 * D, D), lambda i: (0, 0)),
                  pl.BlockSpec((D,), lambda i: (0,))],
        out_specs=row_spec,
        out_shape=jax.ShapeDtypeStruct((N, D), jnp.float32),
    )(x, attnout, Wo, bo, ln2_g, ln2_b, W1, b1, W2, b2)


BM = 256


def _attn_body(q_ref, k_ref, v_ref, m_ref, o_ref):
    q = q_ref[0]
    k = k_ref[0]
    v = v_ref[0]
    s = jax.lax.dot_general(q, k, (((1,), (1,)), ((), ())),
                            preferred_element_type=jnp.float32) * (1.0 / 8.0)
    mx = jnp.max(s, axis=1, keepdims=True)
    w = jnp.exp(s - mx) * m_ref[...]
    denom = jnp.sum(w, axis=1, keepdims=True) + 1e-9
    p = w / denom
    o_ref[0] = jax.lax.dot_general(p, v, (((1,), (0,)), ((), ())),
                                   preferred_element_type=jnp.float32)


def _attn(qh, kh, vh, mask):
    grid = (N // BM, H)
    return pl.pallas_call(
        _attn_body,
        grid=grid,
        in_specs=[pl.BlockSpec((1, BM, DH), lambda i, h: (h, i, 0)),
                  pl.BlockSpec((1, N, DH), lambda i, h: (h, 0, 0)),
                  pl.BlockSpec((1, N, DH), lambda i, h: (h, 0, 0)),
                  pl.BlockSpec((BM, N), lambda i, h: (i, 0))],
        out_specs=pl.BlockSpec((1, BM, DH), lambda i, h: (h, i, 0)),
        out_shape=jax.ShapeDtypeStruct((H, N, DH), jnp.float32),
    )(qh, kh, vh, mask)


def kernel(x, batch_mask_indices, batch_mask_values, attention_mask_indices, attention_mask_values, ln1_g, ln1_b, Wq, bq, Wk, bk, Wv, bv, Wo, bo, ln2_g, ln2_b, W1, b1, W2, b2):
    rows = attention_mask_indices[0]
    cols = attention_mask_indices[1]
    mask = jnp.zeros((N, N), jnp.float32).at[rows, cols].add(
        batch_mask_values * attention_mask_values)
    q, k, v = _qkv(x, ln1_g, ln1_b, Wq, bq, Wk, bk, Wv, bv)
    qh = q.reshape(N, H, DH).transpose(1, 0, 2)
    kh = k.reshape(N, H, DH).transpose(1, 0, 2)
    vh = v.reshape(N, H, DH).transpose(1, 0, 2)
    out = _attn(qh, kh, vh, mask).transpose(1, 0, 2)
    return _tail(x, out.reshape(N, D), Wo, bo, ln2_g, ln2_b, W1, b1, W2, b2)
